# Initial kernel scaffold; baseline (speedup 1.0000x reference)
import functools

import jax
import jax.numpy as jnp
from jax import lax
from jax.experimental import pallas as pl
from jax.experimental.pallas import tpu as pltpu
from jax.experimental.pallas import tpu_sc as plsc

N_NODES = 10000
N_EDGES = 320000
D = 128
ALPHA = 0.5

NS = 16
CHUNK = 80
EDGES_PER_TILE = N_EDGES // NS
NCHUNK = EDGES_PER_TILE // CHUNK
ROWS_PER_TILE = N_NODES // NS
HIST_W = 16

_mesh = plsc.VectorSubcoreMesh(core_axis_name="c", subcore_axis_name="s")


@functools.partial(
    pl.kernel,
    out_type=(
        jax.ShapeDtypeStruct((N_NODES, HIST_W), jnp.float32),
        jax.ShapeDtypeStruct((N_NODES, HIST_W), jnp.float32),
    ),
    mesh=_mesh,
    scratch_types=[
        pltpu.VMEM((NCHUNK, CHUNK), jnp.int32),
        pltpu.VMEM((CHUNK, HIST_W), jnp.float32),
        pltpu.VMEM_SHARED((N_NODES, HIST_W), jnp.float32),
    ],
)
def _degree_sc(row_hbm, col_hbm, zeros_hbm, ones_hbm, degf_hbm, degb_hbm,
               idx_v, ones_v, hist):
    core = lax.axis_index("c")
    tile = lax.axis_index("s")
    rows = pl.ds(tile * ROWS_PER_TILE, ROWS_PER_TILE)

    pltpu.sync_copy(zeros_hbm.at[rows], hist.at[rows])
    pltpu.sync_copy(ones_hbm, ones_v)

    @pl.when(core == 0)
    def _():
        pltpu.sync_copy(row_hbm.at[tile], idx_v)

    @pl.when(core == 1)
    def _():
        pltpu.sync_copy(col_hbm.at[tile], idx_v)

    plsc.subcore_barrier()

    @pl.loop(0, NCHUNK)
    def _(c):
        pltpu.sync_copy(ones_v, hist.at[idx_v.at[c]], add=True)

    plsc.subcore_barrier()

    @pl.when(core == 0)
    def _():
        pltpu.sync_copy(hist.at[rows], degf_hbm.at[rows])

    @pl.when(core == 1)
    def _():
        pltpu.sync_copy(hist.at[rows], degb_hbm.at[rows])


@functools.partial(
    pl.kernel,
    out_type=(
        jax.ShapeDtypeStruct((N_NODES, D), jnp.float32),
        jax.ShapeDtypeStruct((N_NODES, D), jnp.float32),
    ),
    mesh=_mesh,
    scratch_types=[
        pltpu.VMEM((NCHUNK, CHUNK), jnp.int32),
        pltpu.VMEM((NCHUNK, CHUNK), jnp.int32),
        pltpu.VMEM((CHUNK, D), jnp.float32),
        pltpu.VMEM((CHUNK, D), jnp.float32),
        pltpu.VMEM_SHARED((N_NODES, D), jnp.float32),
        pltpu.SemaphoreType.DMA,
        pltpu.SemaphoreType.DMA,
    ],
)
def _spmm_sc(row_hbm, col_hbm, xs_hbm, xd_hbm, zeros_hbm, outf_hbm, outb_hbm,
             row_v, col_v, buf_a, buf_b, acc, sem_a, sem_b):
    core = lax.axis_index("c")
    tile = lax.axis_index("s")
    rows = pl.ds(tile * ROWS_PER_TILE, ROWS_PER_TILE)

    pltpu.sync_copy(zeros_hbm.at[rows], acc.at[rows])
    pltpu.sync_copy(row_hbm.at[tile], row_v)
    pltpu.sync_copy(col_hbm.at[tile], col_v)
    plsc.subcore_barrier()

    def run_direction(src_hbm, g_idx, s_idx):
        def issue(c, buf, sem):
            pltpu.make_async_copy(src_hbm.at[g_idx.at[c]], buf, sem).start()

        def wait(c, buf, sem):
            pltpu.make_async_copy(src_hbm.at[g_idx.at[c]], buf, sem).wait()

        issue(0, buf_a, sem_a)

        @pl.loop(0, NCHUNK, step=2)
        def _(c):
            issue(c + 1, buf_b, sem_b)
            wait(c, buf_a, sem_a)
            pltpu.sync_copy(buf_a, acc.at[s_idx.at[c]], add=True)

            @pl.when(c + 2 < NCHUNK)
            def _():
                issue(c + 2, buf_a, sem_a)

            wait(c + 1, buf_b, sem_b)
            pltpu.sync_copy(buf_b, acc.at[s_idx.at[c + 1]], add=True)

    @pl.when(core == 0)
    def _():
        run_direction(xs_hbm, col_v, row_v)

    @pl.when(core == 1)
    def _():
        run_direction(xd_hbm, row_v, col_v)

    plsc.subcore_barrier()

    @pl.when(core == 0)
    def _():
        pltpu.sync_copy(acc.at[rows], outf_hbm.at[rows])

    @pl.when(core == 1)
    def _():
        pltpu.sync_copy(acc.at[rows], outb_hbm.at[rows])


def _matmul_tc(x, w_sd, w_ds):
    def body(x_ref, wsd_ref, wds_ref, xs_ref, xd_ref):
        xb = x_ref[...]
        dn = (((1,), (1,)), ((), ()))
        xs_ref[...] = ALPHA * lax.dot_general(
            xb, wsd_ref[...], dn, preferred_element_type=jnp.float32)
        xd_ref[...] = (1.0 - ALPHA) * lax.dot_general(
            xb, wds_ref[...], dn, preferred_element_type=jnp.float32)

    blk = N_NODES // 10
    return pl.pallas_call(
        body,
        grid=(10,),
        in_specs=[
            pl.BlockSpec((blk, D), lambda i: (i, 0)),
            pl.BlockSpec((D, D), lambda i: (0, 0)),
            pl.BlockSpec((D, D), lambda i: (0, 0)),
        ],
        out_specs=[
            pl.BlockSpec((blk, D), lambda i: (i, 0)),
            pl.BlockSpec((blk, D), lambda i: (i, 0)),
        ],
        out_shape=[jax.ShapeDtypeStruct((N_NODES, D), jnp.float32)] * 2,
    )(x, w_sd, w_ds)


def _dinv(deg_block):
    return jnp.where(deg_block > 0,
                     lax.rsqrt(jnp.maximum(deg_block, 1e-12)),
                     0.0)


def _scale_tc(xs0, xd0, degf, degb):
    def body(xs_ref, xd_ref, df_ref, db_ref, oxs_ref, oxd_ref):
        oxs_ref[...] = _dinv(df_ref[:, 0:1]) * xs_ref[...]
        oxd_ref[...] = _dinv(db_ref[:, 0:1]) * xd_ref[...]

    blk = N_NODES // 10
    return pl.pallas_call(
        body,
        grid=(10,),
        in_specs=[
            pl.BlockSpec((blk, D), lambda i: (i, 0)),
            pl.BlockSpec((blk, D), lambda i: (i, 0)),
            pl.BlockSpec((blk, HIST_W), lambda i: (i, 0)),
            pl.BlockSpec((blk, HIST_W), lambda i: (i, 0)),
        ],
        out_specs=[
            pl.BlockSpec((blk, D), lambda i: (i, 0)),
            pl.BlockSpec((blk, D), lambda i: (i, 0)),
        ],
        out_shape=[jax.ShapeDtypeStruct((N_NODES, D), jnp.float32)] * 2,
    )(xs0, xd0, degf, degb)


def _combine_tc(accf, accb, degf, degb, bsd, bds):
    def body(af_ref, ab_ref, df_ref, db_ref, bsd_ref, bds_ref, o_ref):
        bias = ALPHA * bsd_ref[0:1, :] + (1.0 - ALPHA) * bds_ref[0:1, :]
        o_ref[...] = (_dinv(df_ref[:, 0:1]) * af_ref[...]
                      + _dinv(db_ref[:, 0:1]) * ab_ref[...] + bias)

    blk = N_NODES // 10
    return pl.pallas_call(
        body,
        grid=(10,),
        in_specs=[
            pl.BlockSpec((blk, D), lambda i: (i, 0)),
            pl.BlockSpec((blk, D), lambda i: (i, 0)),
            pl.BlockSpec((blk, HIST_W), lambda i: (i, 0)),
            pl.BlockSpec((blk, HIST_W), lambda i: (i, 0)),
            pl.BlockSpec((8, D), lambda i: (0, 0)),
            pl.BlockSpec((8, D), lambda i: (0, 0)),
        ],
        out_specs=pl.BlockSpec((blk, D), lambda i: (i, 0)),
        out_shape=jax.ShapeDtypeStruct((N_NODES, D), jnp.float32),
    )(accf, accb, degf, degb, bsd, bds)


@jax.jit
def kernel(x, edge_index, W_sd, b_sd, W_ds, b_ds):
    row = edge_index[0].reshape(NS, NCHUNK, CHUNK)
    col = edge_index[1].reshape(NS, NCHUNK, CHUNK)

    zeros_hist = jnp.zeros((N_NODES, HIST_W), jnp.float32)
    ones_chunk = jnp.ones((CHUNK, HIST_W), jnp.float32)
    zeros_acc = jnp.zeros((N_NODES, D), jnp.float32)

    degf, degb = _degree_sc(row, col, zeros_hist, ones_chunk)
    xs0, xd0 = _matmul_tc(x, W_sd, W_ds)
    xs, xd = _scale_tc(xs0, xd0, degf, degb)
    accf, accb = _spmm_sc(row, col, xs, xd, zeros_acc)

    bsd = jnp.broadcast_to(b_sd[None, :], (8, D))
    bds = jnp.broadcast_to(b_ds[None, :], (8, D))
    return _combine_tc(accf, accb, degf, degb, bsd, bds)


# speedup vs baseline: 30.4036x; 30.4036x over previous
import functools

import jax
import jax.numpy as jnp
from jax import lax
from jax.experimental import pallas as pl
from jax.experimental.pallas import tpu as pltpu
from jax.experimental.pallas import tpu_sc as plsc

N_NODES = 10000
N_EDGES = 320000
D = 128
ALPHA = 0.5

NS = 16
CHUNK = 80
EDGES_PER_TILE = N_EDGES // NS
NCHUNK = EDGES_PER_TILE // CHUNK
ROWS_PER_TILE = 624
ROWS_TAIL = N_NODES - NS * ROWS_PER_TILE
HIST_W = 16

_mesh = plsc.VectorSubcoreMesh(core_axis_name="c", subcore_axis_name="s")
_sc_params = pltpu.CompilerParams(use_tc_tiling_on_sc=False)


def _for_tile_rows(tile, fn):
    base = tile * ROWS_PER_TILE

    @pl.loop(0, 7)
    def _(c):
        fn(base + c * CHUNK, CHUNK)

    fn(base + 7 * CHUNK, ROWS_PER_TILE - 7 * CHUNK)

    @pl.when(tile == NS - 1)
    def _():
        fn(NS * ROWS_PER_TILE, ROWS_TAIL)


def _fill_rows(buf, width, value):
    @pl.loop(0, CHUNK)
    def _(i):
        for j in range(width // 16):
            buf[i, pl.ds(j * 16, 16)] = jnp.full((16,), value, jnp.float32)


@functools.partial(
    pl.kernel,
    out_type=(
        jax.ShapeDtypeStruct((N_NODES, HIST_W), jnp.float32),
        jax.ShapeDtypeStruct((N_NODES, HIST_W), jnp.float32),
    ),
    mesh=_mesh,
    scratch_types=[
        pltpu.VMEM((NCHUNK, CHUNK), jnp.int32),
        pltpu.VMEM((CHUNK, HIST_W), jnp.float32),
        pltpu.VMEM((CHUNK, HIST_W), jnp.float32),
        pltpu.VMEM_SHARED((N_NODES, HIST_W), jnp.float32),
    ],
    compiler_params=_sc_params,
)
def _degree_sc(row_hbm, col_hbm, degf_hbm, degb_hbm, idx_v, ones_v, zero_v,
               hist):
    core = lax.axis_index("c")
    tile = lax.axis_index("s")

    _fill_rows(ones_v, HIST_W, 1.0)
    _fill_rows(zero_v, HIST_W, 0.0)
    _for_tile_rows(
        tile,
        lambda off, sz: pltpu.sync_copy(zero_v.at[pl.ds(0, sz)],
                                        hist.at[pl.ds(off, sz)]))

    @pl.when(core == 0)
    def _():
        pltpu.sync_copy(row_hbm.at[tile], idx_v)

    @pl.when(core == 1)
    def _():
        pltpu.sync_copy(col_hbm.at[tile], idx_v)

    plsc.subcore_barrier()

    @pl.loop(0, NCHUNK)
    def _(c):
        pltpu.sync_copy(ones_v, hist.at[idx_v.at[c]], add=True)

    plsc.subcore_barrier()

    def _writeout(out_hbm):
        def fn(off, sz):
            pltpu.sync_copy(hist.at[pl.ds(off, sz)], zero_v.at[pl.ds(0, sz)])
            pltpu.sync_copy(zero_v.at[pl.ds(0, sz)], out_hbm.at[pl.ds(off, sz)])

        _for_tile_rows(tile, fn)

    @pl.when(core == 0)
    def _():
        _writeout(degf_hbm)

    @pl.when(core == 1)
    def _():
        _writeout(degb_hbm)


DH = D // 2


@functools.partial(
    pl.kernel,
    out_type=(
        jax.ShapeDtypeStruct((N_NODES, DH), jnp.float32),
        jax.ShapeDtypeStruct((N_NODES, DH), jnp.float32),
    ),
    mesh=_mesh,
    scratch_types=[
        pltpu.VMEM((NCHUNK, CHUNK), jnp.int32),
        pltpu.VMEM((NCHUNK, CHUNK), jnp.int32),
        pltpu.VMEM((CHUNK, DH), jnp.float32),
        pltpu.VMEM((CHUNK, DH), jnp.float32),
        pltpu.VMEM_SHARED((N_NODES, DH), jnp.float32),
        pltpu.SemaphoreType.DMA,
        pltpu.SemaphoreType.DMA,
    ],
    compiler_params=_sc_params,
)
def _spmm_sc(row_hbm, col_hbm, xs_hbm, xd_hbm, outf_hbm, outb_hbm,
             row_v, col_v, buf_a, buf_b, acc, sem_a, sem_b):
    core = lax.axis_index("c")
    tile = lax.axis_index("s")

    _fill_rows(buf_a, DH, 0.0)
    _for_tile_rows(
        tile,
        lambda off, sz: pltpu.sync_copy(buf_a.at[pl.ds(0, sz)],
                                        acc.at[pl.ds(off, sz)]))
    pltpu.sync_copy(row_hbm.at[tile], row_v)
    pltpu.sync_copy(col_hbm.at[tile], col_v)
    plsc.subcore_barrier()

    def run_direction(src_hbm, g_idx, s_idx):
        def issue(c, buf, sem):
            pltpu.make_async_copy(src_hbm.at[g_idx.at[c]], buf, sem).start()

        def wait(c, buf, sem):
            pltpu.make_async_copy(src_hbm.at[g_idx.at[c]], buf, sem).wait()

        issue(0, buf_a, sem_a)

        @pl.loop(0, NCHUNK, step=2)
        def _(c):
            issue(c + 1, buf_b, sem_b)
            wait(c, buf_a, sem_a)
            pltpu.sync_copy(buf_a, acc.at[s_idx.at[c]], add=True)

            @pl.when(c + 2 < NCHUNK)
            def _():
                issue(c + 2, buf_a, sem_a)

            wait(c + 1, buf_b, sem_b)
            pltpu.sync_copy(buf_b, acc.at[s_idx.at[c + 1]], add=True)

    @pl.when(core == 0)
    def _():
        run_direction(xs_hbm, col_v, row_v)

    @pl.when(core == 1)
    def _():
        run_direction(xd_hbm, row_v, col_v)

    plsc.subcore_barrier()

    def _writeout(out_hbm):
        def fn(off, sz):
            pltpu.sync_copy(acc.at[pl.ds(off, sz)], buf_a.at[pl.ds(0, sz)])
            pltpu.sync_copy(buf_a.at[pl.ds(0, sz)], out_hbm.at[pl.ds(off, sz)])

        _for_tile_rows(tile, fn)

    @pl.when(core == 0)
    def _():
        _writeout(outf_hbm)

    @pl.when(core == 1)
    def _():
        _writeout(outb_hbm)


def _matmul_tc(x, w_sd, w_ds):
    def body(x_ref, wsd_ref, wds_ref, xs_ref, xd_ref):
        xb = x_ref[...]
        dn = (((1,), (1,)), ((), ()))
        xs_ref[...] = ALPHA * lax.dot_general(
            xb, wsd_ref[...], dn, preferred_element_type=jnp.float32)
        xd_ref[...] = (1.0 - ALPHA) * lax.dot_general(
            xb, wds_ref[...], dn, preferred_element_type=jnp.float32)

    blk = N_NODES // 10
    return pl.pallas_call(
        body,
        grid=(10,),
        in_specs=[
            pl.BlockSpec((blk, D), lambda i: (i, 0)),
            pl.BlockSpec((D, D), lambda i: (0, 0)),
            pl.BlockSpec((D, D), lambda i: (0, 0)),
        ],
        out_specs=[
            pl.BlockSpec((blk, D), lambda i: (i, 0)),
            pl.BlockSpec((blk, D), lambda i: (i, 0)),
        ],
        out_shape=[jax.ShapeDtypeStruct((N_NODES, D), jnp.float32)] * 2,
    )(x, w_sd, w_ds)


def _dinv(deg_block):
    return jnp.where(deg_block > 0,
                     lax.rsqrt(jnp.maximum(deg_block, 1e-12)),
                     0.0)


def _scale_tc(xs0, xd0, degf, degb):
    def body(xs_ref, xd_ref, df_ref, db_ref, xsl_ref, xsh_ref, xdl_ref,
             xdh_ref):
        xs = _dinv(df_ref[:, 0:1]) * xs_ref[...]
        xd = _dinv(db_ref[:, 0:1]) * xd_ref[...]
        xsl_ref[...] = xs[:, :DH]
        xsh_ref[...] = xs[:, DH:]
        xdl_ref[...] = xd[:, :DH]
        xdh_ref[...] = xd[:, DH:]

    blk = N_NODES // 10
    return pl.pallas_call(
        body,
        grid=(10,),
        in_specs=[
            pl.BlockSpec((blk, D), lambda i: (i, 0)),
            pl.BlockSpec((blk, D), lambda i: (i, 0)),
            pl.BlockSpec((blk, HIST_W), lambda i: (i, 0)),
            pl.BlockSpec((blk, HIST_W), lambda i: (i, 0)),
        ],
        out_specs=[pl.BlockSpec((blk, DH), lambda i: (i, 0))] * 4,
        out_shape=[jax.ShapeDtypeStruct((N_NODES, DH), jnp.float32)] * 4,
    )(xs0, xd0, degf, degb)


def _combine_tc(afl, afh, abl, abh, degf, degb, bsd, bds):
    def body(afl_ref, afh_ref, abl_ref, abh_ref, df_ref, db_ref, bsd_ref,
             bds_ref, o_ref):
        bias = ALPHA * bsd_ref[0:1, :] + (1.0 - ALPHA) * bds_ref[0:1, :]
        dif = _dinv(df_ref[:, 0:1])
        dib = _dinv(db_ref[:, 0:1])
        af = jnp.concatenate([afl_ref[...], afh_ref[...]], axis=1)
        ab = jnp.concatenate([abl_ref[...], abh_ref[...]], axis=1)
        o_ref[...] = dif * af + dib * ab + bias

    blk = N_NODES // 10
    return pl.pallas_call(
        body,
        grid=(10,),
        in_specs=[
            pl.BlockSpec((blk, DH), lambda i: (i, 0)),
            pl.BlockSpec((blk, DH), lambda i: (i, 0)),
            pl.BlockSpec((blk, DH), lambda i: (i, 0)),
            pl.BlockSpec((blk, DH), lambda i: (i, 0)),
            pl.BlockSpec((blk, HIST_W), lambda i: (i, 0)),
            pl.BlockSpec((blk, HIST_W), lambda i: (i, 0)),
            pl.BlockSpec((8, D), lambda i: (0, 0)),
            pl.BlockSpec((8, D), lambda i: (0, 0)),
        ],
        out_specs=pl.BlockSpec((blk, D), lambda i: (i, 0)),
        out_shape=jax.ShapeDtypeStruct((N_NODES, D), jnp.float32),
    )(afl, afh, abl, abh, degf, degb, bsd, bds)


@jax.jit
def kernel(x, edge_index, W_sd, b_sd, W_ds, b_ds):
    row = edge_index[0].reshape(NS, NCHUNK, CHUNK)
    col = edge_index[1].reshape(NS, NCHUNK, CHUNK)

    degf, degb = _degree_sc(row, col)
    xs0, xd0 = _matmul_tc(x, W_sd, W_ds)
    xsl, xsh, xdl, xdh = _scale_tc(xs0, xd0, degf, degb)
    afl, abl = _spmm_sc(row, col, xsl, xdl)
    afh, abh = _spmm_sc(row, col, xsh, xdh)

    bsd = jnp.broadcast_to(b_sd[None, :], (8, D))
    bds = jnp.broadcast_to(b_ds[None, :], (8, D))
    return _combine_tc(afl, afh, abl, abh, degf, degb, bsd, bds)
